# Initial kernel scaffold; baseline (speedup 1.0000x reference)
import jax
import jax.numpy as jnp
from jax.experimental import pallas as pl


def kernel(x, edge_index, W1, b1, gamma1, beta1, W2, b2, gamma2, beta2, W3, b3):
    raise NotImplementedError("write your pallas kernel here")


# speedup vs baseline: 5.0351x; 5.0351x over previous
import functools

import jax
import jax.numpy as jnp
from jax import lax
from jax.experimental import pallas as pl
from jax.experimental.pallas import tpu as pltpu
from jax.experimental.pallas import tpu_sc as plsc

N = 10000
NP = 10240
E = 160000
EP = 163840
FC = 128
NCORES = 2
NSUB = 16
BATCH = 128
TRASH = NP - 8
PAD_SRC = N
BM = 1280
ROWS_PER_SUB = NP // NSUB
DEG_NB = EP // (NCORES * NSUB) // BATCH
AGG_NB = EP // NSUB // BATCH

_mesh = plsc.VectorSubcoreMesh(core_axis_name="c", subcore_axis_name="s")


def _sc_deg_body(dst_hbm, ones_hbm, zeros_hbm, degp_hbm, didx, ones_v, dacc):
    c = lax.axis_index("c")
    s = lax.axis_index("s")
    wid = c * NSUB + s
    pltpu.sync_copy(dst_hbm.at[pl.ds(wid * DEG_NB, DEG_NB)], didx)
    pltpu.sync_copy(ones_hbm, ones_v)
    pltpu.sync_copy(zeros_hbm, dacc.at[pl.ds(s * ROWS_PER_SUB, ROWS_PER_SUB)])
    plsc.subcore_barrier()

    @pl.loop(0, DEG_NB)
    def _(b):
        pltpu.sync_copy(ones_v, dacc.at[didx.at[b]], add=True)

    plsc.subcore_barrier()
    pltpu.sync_copy(dacc.at[pl.ds(s * ROWS_PER_SUB, ROWS_PER_SUB)],
                    degp_hbm.at[c].at[pl.ds(s * ROWS_PER_SUB, ROWS_PER_SUB)])


def _sc_deg(dst2d, ones16, zeros16):
    return pl.kernel(
        _sc_deg_body,
        out_type=jax.ShapeDtypeStruct((NCORES, NP, 16), jnp.float32),
        mesh=_mesh,
        scratch_types=[
            pltpu.VMEM((DEG_NB, BATCH), jnp.int32),
            pltpu.VMEM((BATCH, 16), jnp.float32),
            pltpu.VMEM_SHARED((NP, 16), jnp.float32),
        ],
    )(dst2d, ones16, zeros16)


def _sc_agg_body(nch, ytab_hbm, src_hbm, dst_hbm, zeros_hbm, agg_hbm,
                 sidx, didx, gbuf, accum):
    c = lax.axis_index("c")
    s = lax.axis_index("s")
    pltpu.sync_copy(src_hbm.at[pl.ds(s * AGG_NB, AGG_NB)], sidx)
    pltpu.sync_copy(dst_hbm.at[pl.ds(s * AGG_NB, AGG_NB)], didx)
    for k in range(nch // NCORES):
        ch = c + NCORES * k
        pltpu.sync_copy(zeros_hbm,
                        accum.at[pl.ds(s * ROWS_PER_SUB, ROWS_PER_SUB)])
        plsc.subcore_barrier()

        @pl.loop(0, AGG_NB)
        def _(b):
            pltpu.sync_copy(ytab_hbm.at[ch].at[sidx.at[b]], gbuf)
            pltpu.sync_copy(gbuf, accum.at[didx.at[b]], add=True)

        plsc.subcore_barrier()
        pltpu.sync_copy(accum.at[pl.ds(s * ROWS_PER_SUB, ROWS_PER_SUB)],
                        agg_hbm.at[ch].at[pl.ds(s * ROWS_PER_SUB, ROWS_PER_SUB)])
        plsc.subcore_barrier()


def _sc_agg(ytab, src2d, dst2d, zrows):
    nch = ytab.shape[0]
    return pl.kernel(
        functools.partial(_sc_agg_body, nch),
        out_type=jax.ShapeDtypeStruct((nch, NP, FC), jnp.float32),
        mesh=_mesh,
        scratch_types=[
            pltpu.VMEM((AGG_NB, BATCH), jnp.int32),
            pltpu.VMEM((AGG_NB, BATCH), jnp.int32),
            pltpu.VMEM((BATCH, FC), jnp.float32),
            pltpu.VMEM_SHARED((NP, FC), jnp.float32),
        ],
    )(ytab, src2d, dst2d, zrows)


def _tc_dinv_body(degp_ref, mask_ref, o_ref):
    deg = degp_ref[0, :, 0:1] + degp_ref[1, :, 0:1] + 1.0
    o_ref[...] = lax.rsqrt(deg) * mask_ref[...]


def _tc_dinv(degp, rowmask):
    return pl.pallas_call(
        _tc_dinv_body,
        grid=(NP // BM,),
        in_specs=[
            pl.BlockSpec((NCORES, BM, 16), lambda i: (0, i, 0)),
            pl.BlockSpec((BM, 1), lambda i: (i, 0)),
        ],
        out_specs=pl.BlockSpec((BM, 1), lambda i: (i, 0)),
        out_shape=jax.ShapeDtypeStruct((NP, 1), jnp.float32),
    )(degp, rowmask)


def _tc_mm_body(dinv_ref, x_ref, w_ref, o_ref):
    acc = jnp.dot(x_ref[...], w_ref[...], preferred_element_type=jnp.float32)
    o_ref[0] = acc * dinv_ref[...]


def _tc_mm(dinv, xp, w):
    k_in, d_out = w.shape
    nch = d_out // FC
    return pl.pallas_call(
        _tc_mm_body,
        grid=(NP // BM, nch),
        in_specs=[
            pl.BlockSpec((BM, 1), lambda i, j: (i, 0)),
            pl.BlockSpec((BM, k_in), lambda i, j: (i, 0)),
            pl.BlockSpec((k_in, FC), lambda i, j: (0, j)),
        ],
        out_specs=pl.BlockSpec((1, BM, FC), lambda i, j: (j, i, 0)),
        out_shape=jax.ShapeDtypeStruct((nch, NP, FC), jnp.float32),
    )(dinv, xp, w)


def _tc_ew_body(bn, agg_ref, y_ref, dinv_ref, b_ref, g_ref, bt_ref, o_ref):
    v = (agg_ref[0] + y_ref[0]) * dinv_ref[...] + b_ref[...]
    if bn:
        inv_s = (1.0 + 1e-5) ** -0.5
        v = v * (g_ref[...] * inv_s) + bt_ref[...]
        v = jnp.maximum(v, 0.0)
    o_ref[...] = v


def _tc_ew(agg, y, dinv, b, gamma, beta, bn):
    nch = agg.shape[0]
    d_out = nch * FC
    b2 = b.reshape(1, d_out)
    g2 = (gamma if bn else b).reshape(1, d_out)
    bt2 = (beta if bn else b).reshape(1, d_out)
    return pl.pallas_call(
        functools.partial(_tc_ew_body, bn),
        grid=(NP // BM, nch),
        in_specs=[
            pl.BlockSpec((1, BM, FC), lambda i, j: (j, i, 0)),
            pl.BlockSpec((1, BM, FC), lambda i, j: (j, i, 0)),
            pl.BlockSpec((BM, 1), lambda i, j: (i, 0)),
            pl.BlockSpec((1, FC), lambda i, j: (0, j)),
            pl.BlockSpec((1, FC), lambda i, j: (0, j)),
            pl.BlockSpec((1, FC), lambda i, j: (0, j)),
        ],
        out_specs=pl.BlockSpec((BM, FC), lambda i, j: (i, j)),
        out_shape=jax.ShapeDtypeStruct((NP, d_out), jnp.float32),
    )(agg, y, dinv, b2, g2, bt2)


def kernel(x, edge_index, W1, b1, gamma1, beta1, W2, b2, gamma2, beta2,
           W3, b3):
    ei = jnp.asarray(edge_index, jnp.int32)
    src = jnp.concatenate(
        [ei[0], jnp.full((EP - E,), PAD_SRC, jnp.int32)]).reshape(EP // BATCH,
                                                                  BATCH)
    dst = jnp.concatenate(
        [ei[1], jnp.full((EP - E,), TRASH, jnp.int32)]).reshape(EP // BATCH,
                                                                BATCH)
    xp = jnp.pad(x, ((0, NP - N), (0, 0)))
    ones16 = jnp.ones((BATCH, 16), jnp.float32)
    zeros16 = jnp.zeros((ROWS_PER_SUB, 16), jnp.float32)
    zrows = jnp.zeros((ROWS_PER_SUB, FC), jnp.float32)
    rowmask = (jnp.arange(NP) < N).astype(jnp.float32).reshape(NP, 1)

    degp = _sc_deg(dst, ones16, zeros16)
    dinv = _tc_dinv(degp, rowmask)

    h = xp
    for (w, b, g, bt, bn) in (
        (W1, b1, gamma1, beta1, True),
        (W2, b2, gamma2, beta2, True),
        (W3, b3, None, None, False),
    ):
        y = _tc_mm(dinv, h, w)
        agg = _sc_agg(y, src, dst, zrows)
        h = _tc_ew(agg, y, dinv, b, g, bt, bn)
    return h[:N]
